# baseline (speedup 1.0000x reference)
import jax
import jax.numpy as jnp
from jax.experimental import pallas as pl
from jax.experimental.pallas import tpu as pltpu
from jax.experimental.pallas import tpu_sc as plsc

_WINDOW = 1024
_TCOLS = 8192


def _table_transpose_kernel(in_ref, out_ref):
    blk = in_ref[...]
    out_ref[...] = (
        blk.reshape(16, _TCOLS // 8, 8).transpose(1, 2, 0).reshape(_TCOLS // 8, 128)
    )


def _out_transpose_kernel(in_ref, out_ref):
    out_ref[0] = in_ref[...].T


def kernel(emb_ids, table):
    bsz, seq = emb_ids.shape
    num_rows, dim = table.shape
    n = bsz * seq
    idx = emb_ids.T.reshape(1, n)

    packed = pl.pallas_call(
        _table_transpose_kernel,
        grid=((num_rows + _TCOLS - 1) // _TCOLS,),
        in_specs=[pl.BlockSpec((dim, _TCOLS), lambda i: (0, i))],
        out_specs=pl.BlockSpec((_TCOLS // 8, 128), lambda i: (i, 0)),
        out_shape=jax.ShapeDtypeStruct((num_rows * dim // 128, 128), table.dtype),
        compiler_params=pltpu.CompilerParams(dimension_semantics=("parallel",)),
    )(table.T)
    tbl = packed.reshape(num_rows, dim)

    mesh = plsc.VectorSubcoreMesh(core_axis_name="core", subcore_axis_name="subcore")

    @pl.kernel(
        out_type=jax.ShapeDtypeStruct((n, dim), table.dtype),
        mesh=mesh,
        compiler_params=pltpu.CompilerParams(use_tc_tiling_on_sc=False),
    )
    def _gather_kernel(x_hbm, i_hbm, o_hbm):
        def body(i_vmem, o_vmem):
            pltpu.sync_copy(x_hbm.at[i_vmem.at[0]], o_vmem)

        pltpu.emit_pipeline(
            body,
            grid=(n // _WINDOW,),
            in_specs=[pl.BlockSpec((1, _WINDOW), index_map=lambda i: (0, i))],
            out_specs=[pl.BlockSpec((_WINDOW, dim), index_map=lambda i: (i, 0))],
            core_axis_name=("core", "subcore"),
            dimension_semantics=(pltpu.PARALLEL,),
        )(i_hbm, o_hbm)

    out = _gather_kernel(tbl, idx)

    out_t = pl.pallas_call(
        _out_transpose_kernel,
        grid=(seq,),
        in_specs=[pl.BlockSpec((bsz, dim), lambda i: (i, 0))],
        out_specs=pl.BlockSpec((1, dim, bsz), lambda i: (i, 0, 0)),
        out_shape=jax.ShapeDtypeStruct((seq, dim, bsz), table.dtype),
        compiler_params=pltpu.CompilerParams(dimension_semantics=("parallel",)),
    )(out)
    return out_t.transpose(2, 0, 1)


# speedup vs baseline: 1.2934x; 1.2934x over previous
import jax
import jax.numpy as jnp
from jax.experimental import pallas as pl
from jax.experimental.pallas import tpu as pltpu
from jax.experimental.pallas import tpu_sc as plsc

_WINDOW = 1024
_TCOLS = 8192


def _table_transpose_kernel(in_ref, out_ref):
    out_ref[...] = in_ref[...].T


def _out_transpose_kernel(in_ref, out_ref):
    out_ref[0] = in_ref[...].T


def kernel(emb_ids, table):
    bsz, seq = emb_ids.shape
    num_rows, dim = table.shape
    n = bsz * seq
    idx = emb_ids.T.reshape(1, n)

    tbl = pl.pallas_call(
        _table_transpose_kernel,
        grid=((num_rows + _TCOLS - 1) // _TCOLS,),
        in_specs=[pl.BlockSpec((dim, _TCOLS), lambda i: (0, i))],
        out_specs=pl.BlockSpec((_TCOLS, dim), lambda i: (i, 0)),
        out_shape=jax.ShapeDtypeStruct((num_rows, dim), table.dtype),
        compiler_params=pltpu.CompilerParams(dimension_semantics=("parallel",)),
    )(table.T)

    mesh = plsc.VectorSubcoreMesh(core_axis_name="core", subcore_axis_name="subcore")

    @pl.kernel(
        out_type=jax.ShapeDtypeStruct((n, dim), table.dtype),
        mesh=mesh,
        compiler_params=pltpu.CompilerParams(use_tc_tiling_on_sc=False),
    )
    def _gather_kernel(x_hbm, i_hbm, o_hbm):
        def body(i_vmem, o_vmem):
            pltpu.sync_copy(x_hbm.at[i_vmem.at[0]], o_vmem)

        pltpu.emit_pipeline(
            body,
            grid=(n // _WINDOW,),
            in_specs=[pl.BlockSpec((1, _WINDOW), index_map=lambda i: (0, i))],
            out_specs=[pl.BlockSpec((_WINDOW, dim), index_map=lambda i: (i, 0))],
            core_axis_name=("core", "subcore"),
            dimension_semantics=(pltpu.PARALLEL,),
        )(i_hbm, o_hbm)

    out = _gather_kernel(tbl, idx)

    out_t = pl.pallas_call(
        _out_transpose_kernel,
        grid=(seq,),
        in_specs=[pl.BlockSpec((bsz, dim), lambda i: (i, 0))],
        out_specs=pl.BlockSpec((1, dim, bsz), lambda i: (i, 0, 0)),
        out_shape=jax.ShapeDtypeStruct((seq, dim, bsz), table.dtype),
        compiler_params=pltpu.CompilerParams(dimension_semantics=("parallel",)),
    )(out)
    return out_t.transpose(2, 0, 1)


# speedup vs baseline: 1.8560x; 1.4349x over previous
import jax
from jax import lax
import jax.numpy as jnp
from jax.experimental import pallas as pl
from jax.experimental.pallas import tpu as pltpu
from jax.experimental.pallas import tpu_sc as plsc

_WINDOW = 1024
_TCOLS = 8192
_TSTEPS = 62


def _table_repack_kernel(in_ref, out_ref):
    out_ref[...] = (
        in_ref[...].reshape(16, _TCOLS // 8, 8).transpose(1, 2, 0).reshape(_TCOLS // 8, 128)
    )


def kernel(emb_ids, table):
    bsz, seq = emb_ids.shape
    num_rows, dim = table.shape
    n = bsz * seq
    idx = emb_ids.T.reshape(1, n)

    tbl = table

    mesh = plsc.VectorSubcoreMesh(core_axis_name="core", subcore_axis_name="subcore")
    steps_per_seq = bsz // _WINDOW

    @pl.kernel(
        out_type=jax.ShapeDtypeStruct((seq, dim, bsz), table.dtype),
        mesh=mesh,
        compiler_params=pltpu.CompilerParams(
            use_tc_tiling_on_sc=False, needs_layout_passes=False
        ),
    )
    def _gather_kernel(x_hbm, i_hbm, o_hbm):
        def body(i_vmem, o_vmem):
            def inner(g_vmem, sem):
                pltpu.async_copy(x_hbm.at[i_vmem.at[0]], g_vmem, sem).wait()
                lane = lax.iota(jnp.int32, 16)

                @pl.loop(0, _WINDOW // 16)
                def _(k):
                    rows = k * 16 + lane
                    for c in range(16):
                        vals = plsc.load_gather(
                            g_vmem, [rows, jnp.full((16,), c, jnp.int32)]
                        )
                        o_vmem[0, c, pl.ds(k * 16, 16)] = vals

            pl.run_scoped(
                inner,
                g_vmem=pltpu.VMEM((_WINDOW, dim), table.dtype),
                sem=pltpu.SemaphoreType.DMA,
            )

        pltpu.emit_pipeline(
            body,
            grid=(n // _WINDOW,),
            in_specs=[pl.BlockSpec((1, _WINDOW), index_map=lambda i: (0, i))],
            out_specs=[
                pl.BlockSpec(
                    (1, dim, _WINDOW),
                    index_map=lambda i: (i // steps_per_seq, 0, i % steps_per_seq),
                )
            ],
            core_axis_name=("core", "subcore"),
            dimension_semantics=(pltpu.PARALLEL,),
        )(i_hbm, o_hbm)

    out = _gather_kernel(tbl, idx)
    return out.transpose(2, 0, 1)


# speedup vs baseline: 1.9979x; 1.0765x over previous
import jax
from jax import lax
import jax.numpy as jnp
from jax.experimental import pallas as pl
from jax.experimental.pallas import tpu as pltpu
from jax.experimental.pallas import tpu_sc as plsc

_W = 1024
_NW = 32


def kernel(emb_ids, table):
    bsz, seq = emb_ids.shape
    num_rows, dim = table.shape
    n = bsz * seq
    chunks = n // _W
    per_w = chunks // _NW
    steps_per_seq = bsz // _W
    idx = emb_ids.T.reshape(1, n)

    mesh = plsc.VectorSubcoreMesh(core_axis_name="core", subcore_axis_name="subcore")

    @pl.kernel(
        out_type=jax.ShapeDtypeStruct((seq, dim, bsz), table.dtype),
        mesh=mesh,
        scratch_types=[
            pltpu.VMEM((per_w * _W,), jnp.int32),
            pltpu.VMEM((2, _W, dim), table.dtype),
            pltpu.VMEM((2, dim, _W), table.dtype),
            pltpu.SemaphoreType.DMA,
            pltpu.SemaphoreType.DMA((2,)),
            pltpu.SemaphoreType.DMA((2,)),
        ],
        compiler_params=pltpu.CompilerParams(
            use_tc_tiling_on_sc=False, needs_layout_passes=False
        ),
    )
    def _gather_kernel(x_hbm, i_hbm, o_hbm, ids_v, g_v, t_v, s_i, s_g, s_o):
        wid = lax.axis_index("subcore") * 2 + lax.axis_index("core")
        base = wid * per_w
        pltpu.async_copy(
            i_hbm.at[0, pl.ds(base * _W, per_w * _W)], ids_v, s_i
        ).wait()
        lane = lax.iota(jnp.int32, 16)

        def gather_start(k):
            b = k % 2
            return pltpu.async_copy(
                x_hbm.at[ids_v.at[pl.ds(k * _W, _W)]], g_v.at[b], s_g.at[b]
            )

        def transpose(b):
            @pl.loop(0, _W // 16)
            def _(kk):
                rows = kk * 16 + lane
                for c in range(dim):
                    t_v[b, c, pl.ds(kk * 16, 16)] = plsc.load_gather(
                        g_v.at[b], [rows, jnp.full((16,), c, jnp.int32)]
                    )

        def out_start(k):
            b = k % 2
            cid = base + k
            l = cid // steps_per_seq
            b0 = (cid % steps_per_seq) * _W
            return pltpu.async_copy(
                t_v.at[b], o_hbm.at[l, :, pl.ds(b0, _W)], s_o.at[b]
            )

        g_h = {0: gather_start(0)}
        o_h = {}
        for k in range(per_w):
            if k + 1 < per_w:
                g_h[(k + 1) % 2] = gather_start(k + 1)
            g_h[k % 2].wait()
            if k >= 2:
                o_h[k % 2].wait()
            transpose(k % 2)
            o_h[k % 2] = out_start(k)
        o_h[(per_w - 1) % 2].wait()
        o_h[(per_w - 2) % 2].wait()

    out = _gather_kernel(table, idx)
    return out.transpose(2, 0, 1)
